# Initial kernel scaffold; baseline (speedup 1.0000x reference)
import jax
import jax.numpy as jnp
from jax.experimental import pallas as pl


def kernel(score, bbox_reg, center_reg, hwl_reg, alpha_logit, alpha_reg, proposals_left, proposals_right):
    raise NotImplementedError("write your pallas kernel here")


# speedup vs baseline: 300.6650x; 300.6650x over previous
import math

import jax
import jax.numpy as jnp
from jax.experimental import pallas as pl

_N = 5000
_NC = 7
_NBIN = 10
_R, _L = 8, 640
_NP = _R * _L
_DET = 100
_NEG = -1.0e30
_CLIP = math.log(1000.0 / 16.0)
_IMG_W = 1280.0
_IMG_H = 384.0
_SCORE_T = 0.05
_NMS_T = 0.5


def _decode(px1, py1, px2, py2, dxr, dyr, dwr, dhr, img_w, img_h):
    w = px2 - px1 + 1.0
    h = py2 - py1 + 1.0
    cx = px1 + 0.5 * w
    cy = py1 + 0.5 * h
    dx = dxr * 0.1
    dy = dyr * 0.1
    dw = jnp.minimum(dwr * 0.2, _CLIP)
    dh = jnp.minimum(dhr * 0.2, _CLIP)
    pcx = dx * w + cx
    pcy = dy * h + cy
    pw = jnp.exp(dw) * w
    ph = jnp.exp(dh) * h
    x1 = jnp.clip(pcx - 0.5 * pw, 0.0, img_w - 1.0)
    y1 = jnp.clip(pcy - 0.5 * ph, 0.0, img_h - 1.0)
    x2 = jnp.clip(pcx + 0.5 * pw - 1.0, 0.0, img_w - 1.0)
    y2 = jnp.clip(pcy + 0.5 * ph - 1.0, 0.0, img_h - 1.0)
    return x1, y1, x2, y2, w, h, cx, cy


def _body(feats_ref, base_ref, pl_ref, pr_ref, cl_ref, cr_ref, dm_ref,
          rot_ref, prob_ref, kept_ref):
    j = pl.program_id(0)
    row = jax.lax.broadcasted_iota(jnp.int32, (_R, _L), 0)
    col = jax.lax.broadcasted_iota(jnp.int32, (_R, _L), 1)
    lane = row * _L + col
    in_range = lane < _N

    smax = base_ref[8]
    for r in range(9, 16):
        smax = jnp.maximum(smax, base_ref[r])
    ssum = jnp.zeros((_R, _L), jnp.float32)
    num = jnp.zeros((_R, _L), jnp.float32)
    for r in range(8, 16):
        e = jnp.exp(base_ref[r] - smax)
        ssum = ssum + e
        if r >= 9:
            num = jnp.where(j == (r - 9), e, num)
    sj = num / ssum
    prob_ref[0, 0] = sj

    amax = base_ref[16]
    for r in range(17, 26):
        amax = jnp.maximum(amax, base_ref[r])
    label = jnp.zeros((_R, _L), jnp.int32)
    for r in range(25, 15, -1):
        label = jnp.where(base_ref[r] == amax, jnp.int32(r - 16), label)
    res = jnp.zeros((_R, _L), jnp.float32)
    for r in range(16, 26):
        res = jnp.where(label == (r - 16), base_ref[r + 10], res)
    rot_ref[0] = (label.astype(jnp.float32) * (2.0 * math.pi / _NBIN)
                  - math.pi + res)

    lx1, ly1, lx2, ly2, lw, lh, lcx, lcy = _decode(
        base_ref[0], base_ref[1], base_ref[2], base_ref[3],
        feats_ref[0, 0], feats_ref[0, 1], feats_ref[0, 2], feats_ref[0, 3],
        _IMG_W, _IMG_H)
    rx1, ry1, rx2, ry2, rw, rh, rcx, rcy = _decode(
        base_ref[4], base_ref[5], base_ref[6], base_ref[7],
        feats_ref[0, 4], feats_ref[0, 5], feats_ref[0, 6], feats_ref[0, 7],
        _IMG_W, _IMG_H)
    pl_ref[0, 0] = lx1
    pl_ref[0, 1] = ly1
    pl_ref[0, 2] = lx2
    pl_ref[0, 3] = ly2
    pr_ref[0, 0] = rx1
    pr_ref[0, 1] = ry1
    pr_ref[0, 2] = rx2
    pr_ref[0, 3] = ry2

    cl_ref[0, 0] = feats_ref[0, 8] * lw + lcx
    cl_ref[0, 1] = feats_ref[0, 9] * lh + lcy
    cr_ref[0, 0] = feats_ref[0, 10] * rw + rcx
    cr_ref[0, 1] = feats_ref[0, 11] * rh + rcy
    dm_ref[0, 0] = jnp.exp(feats_ref[0, 12])
    dm_ref[0, 1] = jnp.exp(feats_ref[0, 13])
    dm_ref[0, 2] = jnp.exp(feats_ref[0, 14])

    area_l = (lx2 - lx1 + 1.0) * (ly2 - ly1 + 1.0)
    area_r = (rx2 - rx1 + 1.0) * (ry2 - ry1 + 1.0)
    alive0 = jnp.where((sj > _SCORE_T) & in_range, 1.0, 0.0)

    def body(_, carry):
        alive, keep = carry
        masked = jnp.where(alive > 0.5, sj, _NEG)
        m = jnp.max(masked)
        active = m > 0.5 * _NEG
        idx = jnp.min(jnp.where(masked == m, lane, jnp.int32(2 ** 30)))
        sel = jnp.where((lane == idx) & active, 1.0, 0.0)
        keep = jnp.maximum(keep, sel)

        def pick(v):
            return jnp.sum(sel * v)

        bx1, by1, bx2, by2 = pick(lx1), pick(ly1), pick(lx2), pick(ly2)
        qx1, qy1, qx2, qy2 = pick(rx1), pick(ry1), pick(rx2), pick(ry2)
        iw = jnp.maximum(jnp.minimum(lx2, bx2) - jnp.maximum(lx1, bx1) + 1.0,
                         0.0)
        ih = jnp.maximum(jnp.minimum(ly2, by2) - jnp.maximum(ly1, by1) + 1.0,
                         0.0)
        inter = iw * ih
        ba = (bx2 - bx1 + 1.0) * (by2 - by1 + 1.0)
        iou_l = inter / (area_l + ba - inter)
        iw = jnp.maximum(jnp.minimum(rx2, qx2) - jnp.maximum(rx1, qx1) + 1.0,
                         0.0)
        ih = jnp.maximum(jnp.minimum(ry2, qy2) - jnp.maximum(ry1, qy1) + 1.0,
                         0.0)
        inter = iw * ih
        qa = (qx2 - qx1 + 1.0) * (qy2 - qy1 + 1.0)
        iou_r = inter / (area_r + qa - inter)
        iou = jnp.maximum(iou_l, iou_r)
        alive = jnp.where((iou > _NMS_T) & active, 0.0, alive)
        return alive, keep

    _, keep = jax.lax.fori_loop(
        0, _DET, body, (alive0, jnp.zeros((_R, _L), jnp.float32)))
    kept_ref[0, 0] = jnp.where(keep > 0.5, sj, _NEG)


def _pad_t(x):
    k = x.shape[1]
    xt = jnp.pad(x.T, ((0, 0), (0, _NP - _N)))
    return xt.reshape(k, _R, _L)


def kernel(score, bbox_reg, center_reg, hwl_reg, alpha_logit, alpha_reg,
           proposals_left, proposals_right):
    n = _N
    bl = bbox_reg[:, :32].reshape(n, 8, 4)[:, 1:]
    br = bbox_reg[:, 32:].reshape(n, 8, 4)[:, 1:]
    cl = center_reg[:, :16].reshape(n, 8, 2)[:, 1:]
    cr = center_reg[:, 16:].reshape(n, 8, 2)[:, 1:]
    hw = hwl_reg.reshape(n, 8, 3)[:, 1:]
    feats = jnp.concatenate([bl, br, cl, cr, hw], axis=2)
    feats = jnp.pad(feats, ((0, _NP - n), (0, 0), (0, 1)))
    feats = feats.transpose(1, 2, 0).reshape(_NC, 16, _R, _L)

    base = jnp.concatenate(
        [proposals_left, proposals_right, score, alpha_logit, alpha_reg],
        axis=1)
    base = _pad_t(base)

    fdt = jnp.float32
    out_shape = [
        jax.ShapeDtypeStruct((_NC, 4, _R, _L), fdt),
        jax.ShapeDtypeStruct((_NC, 4, _R, _L), fdt),
        jax.ShapeDtypeStruct((_NC, 2, _R, _L), fdt),
        jax.ShapeDtypeStruct((_NC, 2, _R, _L), fdt),
        jax.ShapeDtypeStruct((_NC, 3, _R, _L), fdt),
        jax.ShapeDtypeStruct((1, _R, _L), fdt),
        jax.ShapeDtypeStruct((_NC, 1, _R, _L), fdt),
        jax.ShapeDtypeStruct((_NC, 1, _R, _L), fdt),
    ]
    cspec = lambda b: pl.BlockSpec(b, lambda j: (j,) + (0,) * (len(b) - 1))
    shared = lambda b: pl.BlockSpec(b, lambda j: (0,) * len(b))
    outs = pl.pallas_call(
        _body,
        grid=(_NC,),
        in_specs=[cspec((1, 16, _R, _L)), shared((36, _R, _L))],
        out_specs=[
            cspec((1, 4, _R, _L)), cspec((1, 4, _R, _L)),
            cspec((1, 2, _R, _L)), cspec((1, 2, _R, _L)),
            cspec((1, 3, _R, _L)), shared((1, _R, _L)),
            cspec((1, 1, _R, _L)), cspec((1, 1, _R, _L)),
        ],
        out_shape=out_shape,
    )(feats, base)
    plb, prb, clb, crb, dmb, rotb, probb, keptb = outs

    kept = keptb.reshape(_NC, _NP)[:, :n].reshape(-1)
    _, top_i = jax.lax.top_k(kept, _DET)
    cls = top_i // n + 1
    prop = top_i % n
    ci = cls - 1
    bL = plb.reshape(_NC, 4, _NP)[ci, :, prop]
    bR = prb.reshape(_NC, 4, _NP)[ci, :, prop]
    cL = clb.reshape(_NC, 2, _NP)[ci, :, prop]
    cR = crb.reshape(_NC, 2, _NP)[ci, :, prop]
    dd = dmb.reshape(_NC, 3, _NP)[ci, :, prop]
    rr = rotb.reshape(_NP)[prop]
    ss = probb.reshape(_NC, _NP)[ci, prop]
    out = jnp.concatenate([bL, bR, cL, cR, dd, rr[:, None], ss[:, None]],
                          axis=1)
    return out, cls


# speedup vs baseline: 738.6979x; 2.4569x over previous
import math

import jax
import jax.numpy as jnp
from jax.experimental import pallas as pl

_N = 5000
_NC = 7
_NBIN = 10
_R, _L = 8, 640
_NP = _R * _L
_DET = 100
_NEG = -1.0e30
_CLIP = math.log(1000.0 / 16.0)
_IMG_W = 1280.0
_IMG_H = 384.0
_SCORE_T = 0.05
_NMS_T = 0.5


def _decode(px1, py1, px2, py2, dxr, dyr, dwr, dhr, img_w, img_h):
    w = px2 - px1 + 1.0
    h = py2 - py1 + 1.0
    cx = px1 + 0.5 * w
    cy = py1 + 0.5 * h
    dx = dxr * 0.1
    dy = dyr * 0.1
    dw = jnp.minimum(dwr * 0.2, _CLIP)
    dh = jnp.minimum(dhr * 0.2, _CLIP)
    pcx = dx * w + cx
    pcy = dy * h + cy
    pw = jnp.exp(dw) * w
    ph = jnp.exp(dh) * h
    x1 = jnp.clip(pcx - 0.5 * pw, 0.0, img_w - 1.0)
    y1 = jnp.clip(pcy - 0.5 * ph, 0.0, img_h - 1.0)
    x2 = jnp.clip(pcx + 0.5 * pw - 1.0, 0.0, img_w - 1.0)
    y2 = jnp.clip(pcy + 0.5 * ph - 1.0, 0.0, img_h - 1.0)
    return x1, y1, x2, y2, w, h, cx, cy


def _body(feats_ref, base_ref, pl_ref, pr_ref, cl_ref, cr_ref, dm_ref,
          rot_ref, prob_ref, kept_ref):
    row = jax.lax.broadcasted_iota(jnp.int32, (1, _R, _L), 1)
    col = jax.lax.broadcasted_iota(jnp.int32, (1, _R, _L), 2)
    lane = row * _L + col
    in_range = lane < _N

    smax = base_ref[8]
    for r in range(9, 16):
        smax = jnp.maximum(smax, base_ref[r])
    es = [jnp.exp(base_ref[r] - smax) for r in range(8, 16)]
    ssum = es[0]
    for e in es[1:]:
        ssum = ssum + e
    sj = jnp.stack(es[1:], axis=0) / ssum[None]
    prob_ref[:, 0] = sj

    amax = base_ref[16]
    for r in range(17, 26):
        amax = jnp.maximum(amax, base_ref[r])
    label = jnp.zeros((_R, _L), jnp.int32)
    for r in range(25, 15, -1):
        label = jnp.where(base_ref[r] == amax, jnp.int32(r - 16), label)
    res = jnp.zeros((_R, _L), jnp.float32)
    for r in range(16, 26):
        res = jnp.where(label == (r - 16), base_ref[r + 10], res)
    rot_ref[0] = (label.astype(jnp.float32) * (2.0 * math.pi / _NBIN)
                  - math.pi + res)

    f = feats_ref[...]
    lx1, ly1, lx2, ly2, lw, lh, lcx, lcy = _decode(
        base_ref[0][None], base_ref[1][None], base_ref[2][None],
        base_ref[3][None],
        f[:, 0], f[:, 1], f[:, 2], f[:, 3], _IMG_W, _IMG_H)
    rx1, ry1, rx2, ry2, rw, rh, rcx, rcy = _decode(
        base_ref[4][None], base_ref[5][None], base_ref[6][None],
        base_ref[7][None],
        f[:, 4], f[:, 5], f[:, 6], f[:, 7], _IMG_W, _IMG_H)
    pl_ref[:, 0] = lx1
    pl_ref[:, 1] = ly1
    pl_ref[:, 2] = lx2
    pl_ref[:, 3] = ly2
    pr_ref[:, 0] = rx1
    pr_ref[:, 1] = ry1
    pr_ref[:, 2] = rx2
    pr_ref[:, 3] = ry2

    cl_ref[:, 0] = f[:, 8] * lw + lcx
    cl_ref[:, 1] = f[:, 9] * lh + lcy
    cr_ref[:, 0] = f[:, 10] * rw + rcx
    cr_ref[:, 1] = f[:, 11] * rh + rcy
    dm_ref[:, 0] = jnp.exp(f[:, 12])
    dm_ref[:, 1] = jnp.exp(f[:, 13])
    dm_ref[:, 2] = jnp.exp(f[:, 14])

    area_l = (lx2 - lx1 + 1.0) * (ly2 - ly1 + 1.0)
    area_r = (rx2 - rx1 + 1.0) * (ry2 - ry1 + 1.0)
    alive0 = jnp.where((sj > _SCORE_T) & in_range, 1.0, 0.0)

    def body(_, carry):
        alive, keep = carry
        masked = jnp.where(alive > 0.5, sj, _NEG)
        m = jnp.max(masked, axis=(1, 2), keepdims=True)
        active = m > 0.5 * _NEG
        idx = jnp.min(jnp.where(masked == m, lane, jnp.int32(2 ** 30)),
                      axis=(1, 2), keepdims=True)
        sel = jnp.where((lane == idx) & active, 1.0, 0.0)
        keep = jnp.maximum(keep, sel)

        def pick(v):
            return jnp.sum(sel * v, axis=(1, 2), keepdims=True)

        bx1, by1, bx2, by2 = pick(lx1), pick(ly1), pick(lx2), pick(ly2)
        qx1, qy1, qx2, qy2 = pick(rx1), pick(ry1), pick(rx2), pick(ry2)
        iw = jnp.maximum(jnp.minimum(lx2, bx2) - jnp.maximum(lx1, bx1) + 1.0,
                         0.0)
        ih = jnp.maximum(jnp.minimum(ly2, by2) - jnp.maximum(ly1, by1) + 1.0,
                         0.0)
        inter = iw * ih
        ba = (bx2 - bx1 + 1.0) * (by2 - by1 + 1.0)
        iou_l = inter / (area_l + ba - inter)
        iw = jnp.maximum(jnp.minimum(rx2, qx2) - jnp.maximum(rx1, qx1) + 1.0,
                         0.0)
        ih = jnp.maximum(jnp.minimum(ry2, qy2) - jnp.maximum(ry1, qy1) + 1.0,
                         0.0)
        inter = iw * ih
        qa = (qx2 - qx1 + 1.0) * (qy2 - qy1 + 1.0)
        iou_r = inter / (area_r + qa - inter)
        iou = jnp.maximum(iou_l, iou_r)
        alive = jnp.where((iou > _NMS_T) & active, 0.0, alive)
        return alive, keep

    _, keep = jax.lax.fori_loop(
        0, _DET, body, (alive0, jnp.zeros((_NC, _R, _L), jnp.float32)))
    kept_ref[:, 0] = jnp.where(keep > 0.5, sj, _NEG)


def _pad_t(x):
    k = x.shape[1]
    xt = jnp.pad(x.T, ((0, 0), (0, _NP - _N)))
    return xt.reshape(k, _R, _L)


def kernel(score, bbox_reg, center_reg, hwl_reg, alpha_logit, alpha_reg,
           proposals_left, proposals_right):
    n = _N
    bl = bbox_reg[:, :32].reshape(n, 8, 4)[:, 1:]
    br = bbox_reg[:, 32:].reshape(n, 8, 4)[:, 1:]
    cl = center_reg[:, :16].reshape(n, 8, 2)[:, 1:]
    cr = center_reg[:, 16:].reshape(n, 8, 2)[:, 1:]
    hw = hwl_reg.reshape(n, 8, 3)[:, 1:]
    feats = jnp.concatenate([bl, br, cl, cr, hw], axis=2)
    feats = jnp.pad(feats, ((0, _NP - n), (0, 0), (0, 1)))
    feats = feats.transpose(1, 2, 0).reshape(_NC, 16, _R, _L)

    base = jnp.concatenate(
        [proposals_left, proposals_right, score, alpha_logit, alpha_reg],
        axis=1)
    base = _pad_t(base)

    fdt = jnp.float32
    out_shape = [
        jax.ShapeDtypeStruct((_NC, 4, _R, _L), fdt),
        jax.ShapeDtypeStruct((_NC, 4, _R, _L), fdt),
        jax.ShapeDtypeStruct((_NC, 2, _R, _L), fdt),
        jax.ShapeDtypeStruct((_NC, 2, _R, _L), fdt),
        jax.ShapeDtypeStruct((_NC, 3, _R, _L), fdt),
        jax.ShapeDtypeStruct((1, _R, _L), fdt),
        jax.ShapeDtypeStruct((_NC, 1, _R, _L), fdt),
        jax.ShapeDtypeStruct((_NC, 1, _R, _L), fdt),
    ]
    outs = pl.pallas_call(_body, out_shape=out_shape)(feats, base)
    plb, prb, clb, crb, dmb, rotb, probb, keptb = outs

    kept = keptb.reshape(_NC, _NP)[:, :n].reshape(-1)
    _, top_i = jax.lax.top_k(kept, _DET)
    cls = top_i // n + 1
    prop = top_i % n
    ci = cls - 1
    bL = plb.reshape(_NC, 4, _NP)[ci, :, prop]
    bR = prb.reshape(_NC, 4, _NP)[ci, :, prop]
    cL = clb.reshape(_NC, 2, _NP)[ci, :, prop]
    cR = crb.reshape(_NC, 2, _NP)[ci, :, prop]
    dd = dmb.reshape(_NC, 3, _NP)[ci, :, prop]
    rr = rotb.reshape(_NP)[prop]
    ss = probb.reshape(_NC, _NP)[ci, prop]
    out = jnp.concatenate([bL, bR, cL, cR, dd, rr[:, None], ss[:, None]],
                          axis=1)
    return out, cls


# speedup vs baseline: 741.8861x; 1.0043x over previous
import math

import jax
import jax.numpy as jnp
from jax.experimental import pallas as pl

_N = 5000
_NC = 7
_NBIN = 10
_R, _L = 8, 640
_NP = _R * _L
_DET = 100
_NEG = -1.0e30
_CLIP = math.log(1000.0 / 16.0)
_IMG_W = 1280.0
_IMG_H = 384.0
_SCORE_T = 0.05
_NMS_T = 0.5


def _decode(px1, py1, px2, py2, dxr, dyr, dwr, dhr, img_w, img_h):
    w = px2 - px1 + 1.0
    h = py2 - py1 + 1.0
    cx = px1 + 0.5 * w
    cy = py1 + 0.5 * h
    dx = dxr / 10.0
    dy = dyr / 10.0
    dw = jnp.minimum(dwr / 5.0, _CLIP)
    dh = jnp.minimum(dhr / 5.0, _CLIP)
    pcx = dx * w + cx
    pcy = dy * h + cy
    pw = jnp.exp(dw) * w
    ph = jnp.exp(dh) * h
    x1 = jnp.clip(pcx - 0.5 * pw, 0.0, img_w - 1.0)
    y1 = jnp.clip(pcy - 0.5 * ph, 0.0, img_h - 1.0)
    x2 = jnp.clip(pcx + 0.5 * pw - 1.0, 0.0, img_w - 1.0)
    y2 = jnp.clip(pcy + 0.5 * ph - 1.0, 0.0, img_h - 1.0)
    return x1, y1, x2, y2, w, h, cx, cy


def _body(feats_ref, base_ref, pl_ref, pr_ref, cl_ref, cr_ref, dm_ref,
          rot_ref, prob_ref, kept_ref):
    row = jax.lax.broadcasted_iota(jnp.int32, (1, _R, _L), 1)
    col = jax.lax.broadcasted_iota(jnp.int32, (1, _R, _L), 2)
    lane = row * _L + col
    in_range = lane < _N

    smax = base_ref[8]
    for r in range(9, 16):
        smax = jnp.maximum(smax, base_ref[r])
    es = [jnp.exp(base_ref[r] - smax) for r in range(8, 16)]
    ssum = es[0]
    for e in es[1:]:
        ssum = ssum + e
    sj = jnp.stack(es[1:], axis=0) / ssum[None]
    prob_ref[:, 0] = sj

    amax = base_ref[16]
    for r in range(17, 26):
        amax = jnp.maximum(amax, base_ref[r])
    label = jnp.zeros((_R, _L), jnp.int32)
    for r in range(25, 15, -1):
        label = jnp.where(base_ref[r] == amax, jnp.int32(r - 16), label)
    res = jnp.zeros((_R, _L), jnp.float32)
    for r in range(16, 26):
        res = jnp.where(label == (r - 16), base_ref[r + 10], res)
    rot_ref[0] = (label.astype(jnp.float32) * (2.0 * math.pi / _NBIN)
                  - math.pi + res)

    f = feats_ref[...]
    lx1, ly1, lx2, ly2, lw, lh, lcx, lcy = _decode(
        base_ref[0][None], base_ref[1][None], base_ref[2][None],
        base_ref[3][None],
        f[:, 0], f[:, 1], f[:, 2], f[:, 3], _IMG_W, _IMG_H)
    rx1, ry1, rx2, ry2, rw, rh, rcx, rcy = _decode(
        base_ref[4][None], base_ref[5][None], base_ref[6][None],
        base_ref[7][None],
        f[:, 4], f[:, 5], f[:, 6], f[:, 7], _IMG_W, _IMG_H)
    pl_ref[:, 0] = lx1
    pl_ref[:, 1] = ly1
    pl_ref[:, 2] = lx2
    pl_ref[:, 3] = ly2
    pr_ref[:, 0] = rx1
    pr_ref[:, 1] = ry1
    pr_ref[:, 2] = rx2
    pr_ref[:, 3] = ry2

    cl_ref[:, 0] = f[:, 8] * lw + lcx
    cl_ref[:, 1] = f[:, 9] * lh + lcy
    cr_ref[:, 0] = f[:, 10] * rw + rcx
    cr_ref[:, 1] = f[:, 11] * rh + rcy
    dm_ref[:, 0] = jnp.exp(f[:, 12])
    dm_ref[:, 1] = jnp.exp(f[:, 13])
    dm_ref[:, 2] = jnp.exp(f[:, 14])

    area_l = (lx2 - lx1 + 1.0) * (ly2 - ly1 + 1.0)
    area_r = (rx2 - rx1 + 1.0) * (ry2 - ry1 + 1.0)
    alive0 = jnp.where((sj > _SCORE_T) & in_range, 1.0, 0.0)

    def body(_, carry):
        alive, keep = carry
        masked = jnp.where(alive > 0.5, sj, _NEG)
        m = jnp.max(masked, axis=(1, 2), keepdims=True)
        active = m > 0.5 * _NEG
        idx = jnp.min(jnp.where(masked == m, lane, jnp.int32(2 ** 30)),
                      axis=(1, 2), keepdims=True)
        sel = jnp.where((lane == idx) & active, 1.0, 0.0)
        keep = jnp.maximum(keep, sel)

        def pick(v):
            return jnp.sum(sel * v, axis=(1, 2), keepdims=True)

        bx1, by1, bx2, by2 = pick(lx1), pick(ly1), pick(lx2), pick(ly2)
        qx1, qy1, qx2, qy2 = pick(rx1), pick(ry1), pick(rx2), pick(ry2)
        iw = jnp.maximum(jnp.minimum(lx2, bx2) - jnp.maximum(lx1, bx1) + 1.0,
                         0.0)
        ih = jnp.maximum(jnp.minimum(ly2, by2) - jnp.maximum(ly1, by1) + 1.0,
                         0.0)
        inter = iw * ih
        ba = (bx2 - bx1 + 1.0) * (by2 - by1 + 1.0)
        sup_l = inter > _NMS_T * (area_l + ba - inter)
        iw = jnp.maximum(jnp.minimum(rx2, qx2) - jnp.maximum(rx1, qx1) + 1.0,
                         0.0)
        ih = jnp.maximum(jnp.minimum(ry2, qy2) - jnp.maximum(ry1, qy1) + 1.0,
                         0.0)
        inter = iw * ih
        qa = (qx2 - qx1 + 1.0) * (qy2 - qy1 + 1.0)
        sup_r = inter > _NMS_T * (area_r + qa - inter)
        alive = jnp.where((sup_l | sup_r) & active, 0.0, alive)
        return alive, keep

    _, keep = jax.lax.fori_loop(
        0, _DET, body, (alive0, jnp.zeros((_NC, _R, _L), jnp.float32)))
    kept_ref[:, 0] = jnp.where(keep > 0.5, sj, _NEG)


def _pad_t(x):
    k = x.shape[1]
    xt = jnp.pad(x.T, ((0, 0), (0, _NP - _N)))
    return xt.reshape(k, _R, _L)


def kernel(score, bbox_reg, center_reg, hwl_reg, alpha_logit, alpha_reg,
           proposals_left, proposals_right):
    n = _N
    bl = bbox_reg[:, :32].reshape(n, 8, 4)[:, 1:]
    br = bbox_reg[:, 32:].reshape(n, 8, 4)[:, 1:]
    cl = center_reg[:, :16].reshape(n, 8, 2)[:, 1:]
    cr = center_reg[:, 16:].reshape(n, 8, 2)[:, 1:]
    hw = hwl_reg.reshape(n, 8, 3)[:, 1:]
    feats = jnp.concatenate([bl, br, cl, cr, hw], axis=2)
    feats = jnp.pad(feats, ((0, _NP - n), (0, 0), (0, 1)))
    feats = feats.transpose(1, 2, 0).reshape(_NC, 16, _R, _L)

    base = jnp.concatenate(
        [proposals_left, proposals_right, score, alpha_logit, alpha_reg],
        axis=1)
    base = _pad_t(base)

    fdt = jnp.float32
    out_shape = [
        jax.ShapeDtypeStruct((_NC, 4, _R, _L), fdt),
        jax.ShapeDtypeStruct((_NC, 4, _R, _L), fdt),
        jax.ShapeDtypeStruct((_NC, 2, _R, _L), fdt),
        jax.ShapeDtypeStruct((_NC, 2, _R, _L), fdt),
        jax.ShapeDtypeStruct((_NC, 3, _R, _L), fdt),
        jax.ShapeDtypeStruct((1, _R, _L), fdt),
        jax.ShapeDtypeStruct((_NC, 1, _R, _L), fdt),
        jax.ShapeDtypeStruct((_NC, 1, _R, _L), fdt),
    ]
    outs = pl.pallas_call(_body, out_shape=out_shape)(feats, base)
    plb, prb, clb, crb, dmb, rotb, probb, keptb = outs

    kept = keptb.reshape(_NC, _NP)[:, :n].reshape(-1)
    _, top_i = jax.lax.top_k(kept, _DET)
    cls = top_i // n + 1
    prop = top_i % n
    ci = cls - 1
    bL = plb.reshape(_NC, 4, _NP)[ci, :, prop]
    bR = prb.reshape(_NC, 4, _NP)[ci, :, prop]
    cL = clb.reshape(_NC, 2, _NP)[ci, :, prop]
    cR = crb.reshape(_NC, 2, _NP)[ci, :, prop]
    dd = dmb.reshape(_NC, 3, _NP)[ci, :, prop]
    rr = rotb.reshape(_NP)[prop]
    ss = probb.reshape(_NC, _NP)[ci, prop]
    out = jnp.concatenate([bL, bR, cL, cR, dd, rr[:, None], ss[:, None]],
                          axis=1)
    return out, cls
